# Initial kernel scaffold; baseline (speedup 1.0000x reference)
import jax
import jax.numpy as jnp
from jax.experimental import pallas as pl


def kernel(inputs, w, w_mA, w_cAB, w_mB, w_cBA):
    raise NotImplementedError("write your pallas kernel here")


# speedup vs baseline: 10.6861x; 10.6861x over previous
import jax
import jax.numpy as jnp
from jax import lax
from jax.experimental import pallas as pl
from jax.experimental.pallas import tpu as pltpu
from jax.experimental.pallas import tpu_sc as plsc

N = 1000
B = 16384
NC = 2
NS = 16
NW = NC * NS
BPW = B // NW
CHUNK = 128
NCH = BPW // CHUNK
NV = BPW // 16


def _tc_body(wmA_ref, cab_ref, wmB_ref, cba_ref, tA_ref, tB_ref):
    def t_for(wm, c):
        m = jnp.max(c, axis=1)
        lse = jnp.log(jnp.sum(jnp.exp(c - m[:, None]), axis=1)) + m
        mm = jnp.max(wm)
        lse_m = jnp.log(jnp.sum(jnp.exp(wm - mm))) + mm
        return wm - lse_m - lse

    tA_ref[:] = t_for(wmA_ref[:], cab_ref[:])
    tB_ref[:] = t_for(wmB_ref[:], cba_ref[:])


_tc_call = pl.pallas_call(
    _tc_body,
    out_shape=(
        jax.ShapeDtypeStruct((N,), jnp.float32),
        jax.ShapeDtypeStruct((N,), jnp.float32),
    ),
)


def _sc_body(a_hbm, b_hbm, tA_hbm, tB_hbm, wab_hbm, wba_hbm,
             outA_hbm, outB_hbm,
             a_v, b_v, idxA, idxB, idxTA, idxTB, gA, gB, gtA, gtB,
             accA_v, accB_v, sem):
    wid = lax.axis_index("s") * NC + lax.axis_index("c")
    base = wid * BPW
    pltpu.sync_copy(a_hbm.at[pl.ds(base, BPW)], a_v)
    pltpu.sync_copy(b_hbm.at[pl.ds(base, BPW)], b_v)
    for j in range(NV):
        a16 = a_v[pl.ds(16 * j, 16)]
        b16 = b_v[pl.ds(16 * j, 16)]
        idxA[j // 8, pl.ds(16 * (j % 8), 16)] = a16 * N + b16
        idxB[j // 8, pl.ds(16 * (j % 8), 16)] = b16 * N + a16
        idxTA[j // 8, pl.ds(16 * (j % 8), 16)] = a16
        idxTB[j // 8, pl.ds(16 * (j % 8), 16)] = b16
    copies = []
    for c in range(NCH):
        copies.append(pltpu.async_copy(wab_hbm.at[idxA.at[c]], gA.at[c], sem))
        copies.append(pltpu.async_copy(wba_hbm.at[idxB.at[c]], gB.at[c], sem))
        copies.append(pltpu.async_copy(tA_hbm.at[idxTA.at[c]], gtA.at[c], sem))
        copies.append(pltpu.async_copy(tB_hbm.at[idxTB.at[c]], gtB.at[c], sem))
    for cp in copies:
        cp.wait()
    accA = jnp.zeros((16,), jnp.float32)
    accB = jnp.zeros((16,), jnp.float32)
    for j in range(NV):
        r, s = j // 8, pl.ds(16 * (j % 8), 16)
        accA = accA + gA[r, s] + gtA[r, s]
        accB = accB + gB[r, s] + gtB[r, s]
    accA_v[:] = accA
    accB_v[:] = accB
    pltpu.sync_copy(accA_v, outA_hbm.at[wid])
    pltpu.sync_copy(accB_v, outB_hbm.at[wid])


_sc_call = pl.kernel(
    _sc_body,
    out_type=(
        jax.ShapeDtypeStruct((NW, 16), jnp.float32),
        jax.ShapeDtypeStruct((NW, 16), jnp.float32),
    ),
    mesh=plsc.VectorSubcoreMesh(core_axis_name="c", subcore_axis_name="s"),
    scratch_types=(
        pltpu.VMEM((BPW,), jnp.int32),
        pltpu.VMEM((BPW,), jnp.int32),
        pltpu.VMEM((NCH, CHUNK), jnp.int32),
        pltpu.VMEM((NCH, CHUNK), jnp.int32),
        pltpu.VMEM((NCH, CHUNK), jnp.int32),
        pltpu.VMEM((NCH, CHUNK), jnp.int32),
        pltpu.VMEM((NCH, CHUNK), jnp.float32),
        pltpu.VMEM((NCH, CHUNK), jnp.float32),
        pltpu.VMEM((NCH, CHUNK), jnp.float32),
        pltpu.VMEM((NCH, CHUNK), jnp.float32),
        pltpu.VMEM((16,), jnp.float32),
        pltpu.VMEM((16,), jnp.float32),
        pltpu.SemaphoreType.DMA,
    ),
)


def kernel(inputs, w, w_mA, w_cAB, w_mB, w_cBA):
    a = inputs[:, 0]
    b = inputs[:, 1]
    tA, tB = _tc_call(w_mA, w_cAB, w_mB, w_cBA)
    outA, outB = _sc_call(a, b, tA, tB, w_cAB.reshape(-1), w_cBA.reshape(-1))
    S_AB = jnp.sum(outA)
    S_BA = jnp.sum(outB)
    return jnp.logaddexp(jax.nn.log_sigmoid(w) + S_AB,
                         jax.nn.log_sigmoid(-w) + S_BA)


# speedup vs baseline: 11.6418x; 1.0894x over previous
import jax
import jax.numpy as jnp
from jax import lax
from jax.experimental import pallas as pl
from jax.experimental.pallas import tpu as pltpu
from jax.experimental.pallas import tpu_sc as plsc

N = 1000
B = 16384
NC = 2
NS = 16
NW = NC * NS
BPW = B // NW
CHUNK = 128
NCH = BPW // CHUNK
NV = BPW // 16
HBINS = 1024


def _tc_body(wmA_ref, cab_ref, wmB_ref, cba_ref, tA_ref, tB_ref):
    def t_for(wm, c):
        m = jnp.max(c, axis=1)
        lse = jnp.log(jnp.sum(jnp.exp(c - m[:, None]), axis=1)) + m
        mm = jnp.max(wm)
        lse_m = jnp.log(jnp.sum(jnp.exp(wm - mm))) + mm
        return wm - lse_m - lse

    tA_ref[:] = t_for(wmA_ref[:], cab_ref[:])
    tB_ref[:] = t_for(wmB_ref[:], cba_ref[:])


_tc_call = pl.pallas_call(
    _tc_body,
    out_shape=(
        jax.ShapeDtypeStruct((N,), jnp.float32),
        jax.ShapeDtypeStruct((N,), jnp.float32),
    ),
)


def _sc_body(q_hbm, wab_hbm, wba_hbm,
             outA_hbm, outB_hbm, cntA_hbm, cntB_hbm,
             a_v, b_v, idxE, idxO, idxA, idxB, idxTA, idxTB, gA, gB,
             ones_v, zeros_v, accA_v, accB_v, hist_a, hist_b, sem):
    cid = lax.axis_index("c")
    sid = lax.axis_index("s")
    wid = sid * NC + cid
    base = wid * BPW
    lane = jnp.arange(16, dtype=jnp.int32)
    for j in range(NV):
        e = 2 * (base + 16 * j) + 2 * lane
        idxE[j // 8, pl.ds(16 * (j % 8), 16)] = e
        idxO[j // 8, pl.ds(16 * (j % 8), 16)] = e + 1
    for k in range(8):
        ones_v[pl.ds(16 * k, 16)] = jnp.ones((16,), jnp.float32)

    @pl.when(sid == 0)
    def _zero_hist():
        for k in range(HBINS // 16):
            zeros_v[pl.ds(16 * k, 16)] = jnp.zeros((16,), jnp.float32)
        pltpu.sync_copy(zeros_v, hist_a)
        pltpu.sync_copy(zeros_v, hist_b)

    in_cp = []
    for c in range(NCH):
        in_cp.append(pltpu.async_copy(
            q_hbm.at[idxE.at[c]], a_v.at[pl.ds(CHUNK * c, CHUNK)], sem))
        in_cp.append(pltpu.async_copy(
            q_hbm.at[idxO.at[c]], b_v.at[pl.ds(CHUNK * c, CHUNK)], sem))
    for cp in in_cp:
        cp.wait()

    for j in range(NV):
        a16 = a_v[pl.ds(16 * j, 16)]
        b16 = b_v[pl.ds(16 * j, 16)]
        r, s = j // 8, pl.ds(16 * (j % 8), 16)
        idxA[r, s] = a16 * N + b16
        idxB[r, s] = b16 * N + a16
        idxTA[r, s] = a16
        idxTB[r, s] = b16
    pair_cp = []
    for c in range(NCH):
        pair_cp.append(pltpu.async_copy(wab_hbm.at[idxA.at[c]], gA.at[c], sem))
        pair_cp.append(pltpu.async_copy(wba_hbm.at[idxB.at[c]], gB.at[c], sem))

    plsc.subcore_barrier()
    for c in range(NCH):
        pltpu.sync_copy(ones_v, hist_a.at[idxTA.at[c]], add=True)
        pltpu.sync_copy(ones_v, hist_b.at[idxTB.at[c]], add=True)
    plsc.subcore_barrier()

    @pl.when(sid == 0)
    def _write_hist():
        pltpu.sync_copy(hist_a, cntA_hbm.at[cid])
        pltpu.sync_copy(hist_b, cntB_hbm.at[cid])

    for cp in pair_cp:
        cp.wait()
    accA = jnp.zeros((16,), jnp.float32)
    accB = jnp.zeros((16,), jnp.float32)
    for j in range(NV):
        r, s = j // 8, pl.ds(16 * (j % 8), 16)
        accA = accA + gA[r, s]
        accB = accB + gB[r, s]
    accA_v[:] = accA
    accB_v[:] = accB
    pltpu.sync_copy(accA_v, outA_hbm.at[wid])
    pltpu.sync_copy(accB_v, outB_hbm.at[wid])


_sc_call = pl.kernel(
    _sc_body,
    out_type=(
        jax.ShapeDtypeStruct((NW, 16), jnp.float32),
        jax.ShapeDtypeStruct((NW, 16), jnp.float32),
        jax.ShapeDtypeStruct((NC, HBINS), jnp.float32),
        jax.ShapeDtypeStruct((NC, HBINS), jnp.float32),
    ),
    mesh=plsc.VectorSubcoreMesh(core_axis_name="c", subcore_axis_name="s"),
    scratch_types=(
        pltpu.VMEM((BPW,), jnp.int32),
        pltpu.VMEM((BPW,), jnp.int32),
        pltpu.VMEM((NCH, CHUNK), jnp.int32),
        pltpu.VMEM((NCH, CHUNK), jnp.int32),
        pltpu.VMEM((NCH, CHUNK), jnp.int32),
        pltpu.VMEM((NCH, CHUNK), jnp.int32),
        pltpu.VMEM((NCH, CHUNK), jnp.int32),
        pltpu.VMEM((NCH, CHUNK), jnp.int32),
        pltpu.VMEM((NCH, CHUNK), jnp.float32),
        pltpu.VMEM((NCH, CHUNK), jnp.float32),
        pltpu.VMEM((CHUNK,), jnp.float32),
        pltpu.VMEM((HBINS,), jnp.float32),
        pltpu.VMEM((16,), jnp.float32),
        pltpu.VMEM((16,), jnp.float32),
        pltpu.VMEM_SHARED((HBINS,), jnp.float32),
        pltpu.VMEM_SHARED((HBINS,), jnp.float32),
        pltpu.SemaphoreType.DMA,
    ),
)


def _combine_body(w_ref, tA_ref, tB_ref, cntA_ref, cntB_ref, pA_ref, pB_ref,
                  out_ref):
    cA = (cntA_ref[0, :] + cntA_ref[1, :])[0:N]
    cB = (cntB_ref[0, :] + cntB_ref[1, :])[0:N]
    S_AB = jnp.sum(cA * tA_ref[:]) + jnp.sum(pA_ref[:])
    S_BA = jnp.sum(cB * tB_ref[:]) + jnp.sum(pB_ref[:])
    wv = w_ref[:, :]
    la = -jnp.log(1.0 + jnp.exp(-wv))
    l1a = -jnp.log(1.0 + jnp.exp(wv))
    x = la + S_AB
    y = l1a + S_BA
    m = jnp.maximum(x, y)
    out_ref[:, :] = m + jnp.log(jnp.exp(x - m) + jnp.exp(y - m))


_combine_call = pl.pallas_call(
    _combine_body,
    out_shape=jax.ShapeDtypeStruct((1, 1), jnp.float32),
)


def kernel(inputs, w, w_mA, w_cAB, w_mB, w_cBA):
    q = inputs.reshape(-1)
    tA, tB = _tc_call(w_mA, w_cAB, w_mB, w_cBA)
    outA, outB, cntA, cntB = _sc_call(q, w_cAB.reshape(-1), w_cBA.reshape(-1))
    res = _combine_call(jnp.reshape(w, (1, 1)), tA, tB, cntA, cntB, outA, outB)
    return jnp.reshape(res, ())
